# Initial kernel scaffold; baseline (speedup 1.0000x reference)
import jax
import jax.numpy as jnp
from jax.experimental import pallas as pl


def kernel(x, token_embedding, positional_embedding):
    raise NotImplementedError("write your pallas kernel here")


# speedup vs baseline: 4.2245x; 4.2245x over previous
import functools

import jax
import jax.numpy as jnp
from jax import lax
from jax.experimental import pallas as pl
from jax.experimental.pallas import tpu as pltpu
from jax.experimental.pallas import tpu_sc as plsc


def _sc_embed(x4, table, pos, *, NW, n_ch, CH, T, D, L):
    NC = 2
    mesh = plsc.VectorSubcoreMesh(core_axis_name="c", subcore_axis_name="s")
    per_w = T // NW
    H = CH // 2

    @functools.partial(
        pl.kernel,
        mesh=mesh,
        out_type=jax.ShapeDtypeStruct((T, D), jnp.float32),
        scratch_types=[
            pltpu.VMEM((2, H), jnp.int32),
            pltpu.VMEM((CH, D), jnp.float32),
            pltpu.VMEM((L, D), jnp.float32),
            pltpu.SemaphoreType.DMA,
        ],
    )
    def k(x_hbm, tab_hbm, pos_hbm, out_hbm, idx_v, rows_v, pos_v, sem):
        c = lax.axis_index("c")
        s = lax.axis_index("s")
        wid = s * NC + c
        pltpu.sync_copy(pos_hbm, pos_v)

        def chunk_body(g, carry):
            pltpu.sync_copy(x_hbm.at[wid, g], idx_v)
            cp0 = pltpu.async_copy(tab_hbm.at[idx_v.at[0]], rows_v.at[pl.ds(0, H)], sem)
            cp1 = pltpu.async_copy(tab_hbm.at[idx_v.at[1]], rows_v.at[pl.ds(H, H)], sem)
            cp0.wait()
            cp1.wait()

            def add_row(l, carry2):
                for j in range(D // 16):
                    sl = pl.ds(j * 16, 16)
                    rows_v[l, sl] = rows_v[l, sl] + pos_v[l, sl]
                return carry2

            lax.fori_loop(0, CH, add_row, 0)
            pltpu.sync_copy(rows_v, out_hbm.at[pl.ds(wid * per_w + g * CH, CH)])
            return carry

        lax.fori_loop(0, n_ch, chunk_body, 0)

    return k(x4, table, pos)


def kernel(x, token_embedding, positional_embedding):
    B, L = x.shape
    V, D = token_embedding.shape
    T = B * L
    NW = 32
    CH = L
    per_w = T // NW
    n_ch = per_w // CH
    x4 = x.reshape(NW, n_ch, 2, CH // 2).astype(jnp.int32)
    out = _sc_embed(
        x4, token_embedding, positional_embedding,
        NW=NW, n_ch=n_ch, CH=CH, T=T, D=D, L=L,
    )
    return out.reshape(B, L, D)
